# baseline (speedup 1.0000x reference)
import functools

import jax
import jax.numpy as jnp
from jax import lax
from jax.experimental import pallas as pl
from jax.experimental.pallas import tpu as pltpu
from jax.experimental.pallas import tpu_sc as plsc

N = 10000
E = 320000
H = 128

NW = 32
PERW = E // NW
CHM = 80
NCHM = PERW // CHM
NBUF = 3
IO_TILES = 10
IO_ROWS = N // IO_TILES
ZROWS = 40


def _segment_sum_sc(e, idxA):
    mesh = plsc.VectorSubcoreMesh(core_axis_name="c", subcore_axis_name="s")

    @functools.partial(
        pl.kernel,
        out_type=jax.ShapeDtypeStruct((2, N, H), jnp.float32),
        mesh=mesh,
        scratch_types=[
            pltpu.VMEM((NBUF, CHM, H), jnp.float32),
            pltpu.VMEM((NCHM, CHM), jnp.int32),
            pltpu.VMEM_SHARED((N, H), jnp.float32),
        ] + [pltpu.SemaphoreType.DMA] * (2 * NBUF),
    )
    def k(e_hbm, idxA_hbm, out_hbm, e_buf, idx_buf, acc, *sems):
        lsem = sems[:NBUF]
        ssem = sems[NBUF:]
        c = lax.axis_index("c")
        s = lax.axis_index("s")
        wid = s * 2 + c
        base = wid * PERW

        for j in range(NBUF - 1):
            pltpu.async_copy(e_hbm.at[pl.ds(base + j * CHM, CHM)],
                             e_buf.at[j], lsem[j])
        pltpu.sync_copy(idxA_hbm.at[1, wid], idx_buf)

        @pl.when(s < IO_TILES)
        def _():
            z16 = jnp.zeros((16,), jnp.float32)

            def zrow(r, carry):
                for q in range(H // 16):
                    e_buf[2, r, pl.ds(q * 16, 16)] = z16
                return carry

            lax.fori_loop(0, ZROWS, zrow, 0)
            for i in range(IO_ROWS // ZROWS):
                pltpu.sync_copy(
                    e_buf.at[2, pl.ds(0, ZROWS)],
                    acc.at[pl.ds(s * IO_ROWS + i * ZROWS, ZROWS)])
        plsc.subcore_barrier()

        def body(g, carry):
            for kk in range(NBUF):
                j = NBUF * g + kk
                kp = (kk + 2) % NBUF
                pltpu.make_async_copy(
                    e_hbm.at[pl.ds(base, CHM)], e_buf.at[kk], lsem[kk]).wait()
                pltpu.async_copy(e_buf.at[kk], acc.at[idx_buf.at[j]],
                                 ssem[kk], add=True)

                @pl.when(j >= 1)
                def _():
                    pltpu.make_async_copy(
                        e_buf.at[kp], acc.at[idx_buf.at[0]], ssem[kp]).wait()

                pltpu.async_copy(
                    e_hbm.at[pl.ds(base + (j + 2) * CHM, CHM)],
                    e_buf.at[kp], lsem[kp])
            return carry

        G = (NCHM - 2) // NBUF
        lax.fori_loop(0, G, body, 0)
        for kk in range(NCHM - NBUF * G):
            j = NBUF * G + kk
            kp = (kk + 2) % NBUF
            pltpu.make_async_copy(
                e_hbm.at[pl.ds(base, CHM)], e_buf.at[kk], lsem[kk]).wait()
            pltpu.async_copy(e_buf.at[kk], acc.at[idx_buf.at[j]],
                             ssem[kk], add=True)
            pltpu.make_async_copy(
                e_buf.at[kp], acc.at[idx_buf.at[0]], ssem[kp]).wait()
        pltpu.make_async_copy(
            e_buf.at[1], acc.at[idx_buf.at[0]], ssem[1]).wait()

        plsc.subcore_barrier()
        @pl.when(s < IO_TILES)
        def _():
            pltpu.sync_copy(
                acc.at[pl.ds(s * IO_ROWS, IO_ROWS)],
                out_hbm.at[c, pl.ds(s * IO_ROWS, IO_ROWS)])

    return k(e, idxA)


def _dot_nt(a, w):
    return lax.dot_general(a, w, (((1,), (1,)), ((), ())),
                           preferred_element_type=jnp.float32)


def _mlp_body(p_ref, v_ref, w1_ref, b1_ref, w2_ref, b2_ref, wg_ref,
              bg_ref, o_ref):
    shift = jnp.float32(0.6931471805599453)
    su = p_ref[0] + p_ref[1]
    z1 = _dot_nt(su, w1_ref[...]) + b1_ref[...]
    h1 = jnp.maximum(z1, 0.0) + jnp.log1p(jnp.exp(-jnp.abs(z1))) - shift
    h2 = _dot_nt(h1, w2_ref[...]) + b2_ref[...]
    vv = v_ref[...]
    zg = (_dot_nt(vv, wg_ref[:, :H]) + _dot_nt(h2, wg_ref[:, H:])
          + bg_ref[...])
    g = 1.0 / (1.0 + jnp.exp(-zg))
    o_ref[...] = g * vv + (1.0 - g) * h2


def _mlp_update_tc(partials, v, W1, b1, W2, b2, Wg, bg):
    blk = 10000
    grid = N // blk
    full = lambda shape: pl.BlockSpec(shape, lambda i: (0,) * len(shape))
    return pl.pallas_call(
        _mlp_body,
        grid=(grid,),
        in_specs=[
            pl.BlockSpec((2, blk, H), lambda i: (0, i, 0)),
            pl.BlockSpec((blk, H), lambda i: (i, 0)),
            full((H, H)), full((1, H)),
            full((H, H)), full((1, H)),
            full((H, 2 * H)), full((1, H)),
        ],
        out_specs=pl.BlockSpec((blk, H), lambda i: (i, 0)),
        out_shape=jax.ShapeDtypeStruct((N, H), jnp.float32),
    )(partials, v, W1, b1, W2, b2, Wg, bg)


def kernel(v, e, edge_index, W1, b1, W2, b2, Wg, bg):
    ei = edge_index
    if ei.dtype != jnp.int32:
        ei = lax.bitcast_convert_type(ei, jnp.int32)[..., 0]
    idxA = ei.reshape(2, NW, NCHM, CHM)
    partials = _segment_sum_sc(e, idxA)
    return _mlp_update_tc(
        partials, v,
        W1, b1.reshape(1, H),
        W2, b2.reshape(1, H),
        Wg, bg.reshape(1, H),
    )


# speedup vs baseline: 1.0155x; 1.0155x over previous
import functools

import jax
import jax.numpy as jnp
from jax import lax
from jax.experimental import pallas as pl
from jax.experimental.pallas import tpu as pltpu
from jax.experimental.pallas import tpu_sc as plsc

N = 10000
E = 320000
H = 128

NW = 32
PERW = E // NW
CHM = 80
NCHM = PERW // CHM
NBUF = 3
IO_TILES = 10
IO_ROWS = N // IO_TILES
ZROWS = 40


def _segment_sum_sc(e, idxA):
    mesh = plsc.VectorSubcoreMesh(core_axis_name="c", subcore_axis_name="s")

    @functools.partial(
        pl.kernel,
        out_type=jax.ShapeDtypeStruct((2, N, H), jnp.float32),
        mesh=mesh,
        scratch_types=[
            pltpu.VMEM((NBUF, CHM, H), jnp.float32),
            pltpu.VMEM((NCHM, CHM), jnp.int32),
            pltpu.VMEM_SHARED((N, H), jnp.float32),
        ] + [pltpu.SemaphoreType.DMA] * (2 * NBUF),
    )
    def k(e_hbm, idxA_hbm, out_hbm, e_buf, idx_buf, acc, *sems):
        lsem = sems[:NBUF]
        ssem = sems[NBUF:]
        c = lax.axis_index("c")
        s = lax.axis_index("s")
        wid = s * 2 + c
        base = wid * PERW

        for j in range(NBUF - 1):
            pltpu.async_copy(e_hbm.at[pl.ds(base + j * CHM, CHM)],
                             e_buf.at[j], lsem[j])
        pltpu.sync_copy(idxA_hbm.at[1, wid], idx_buf)

        @pl.when(s < IO_TILES)
        def _():
            z16 = jnp.zeros((16,), jnp.float32)

            def zrow(r, carry):
                for q in range(H // 16):
                    e_buf[2, r, pl.ds(q * 16, 16)] = z16
                return carry

            lax.fori_loop(0, ZROWS, zrow, 0)
            for i in range(IO_ROWS // ZROWS):
                pltpu.sync_copy(
                    e_buf.at[2, pl.ds(0, ZROWS)],
                    acc.at[pl.ds(s * IO_ROWS + i * ZROWS, ZROWS)])
        plsc.subcore_barrier()

        def body(g, carry):
            for kk in range(NBUF):
                j = NBUF * g + kk
                kp = (kk + 2) % NBUF
                pltpu.make_async_copy(
                    e_hbm.at[pl.ds(base, CHM)], e_buf.at[kk], lsem[kk]).wait()
                pltpu.async_copy(e_buf.at[kk], acc.at[idx_buf.at[j]],
                                 ssem[kk], add=True)

                @pl.when(j >= 1)
                def _():
                    pltpu.make_async_copy(
                        e_buf.at[kp], acc.at[idx_buf.at[0]], ssem[kp]).wait()

                pltpu.async_copy(
                    e_hbm.at[pl.ds(base + (j + 2) * CHM, CHM)],
                    e_buf.at[kp], lsem[kp])
            return carry

        G = (NCHM - 2) // NBUF
        lax.fori_loop(0, G, body, 0)
        for kk in range(NCHM - NBUF * G):
            j = NBUF * G + kk
            kp = (kk + 2) % NBUF
            pltpu.make_async_copy(
                e_hbm.at[pl.ds(base, CHM)], e_buf.at[kk], lsem[kk]).wait()
            pltpu.async_copy(e_buf.at[kk], acc.at[idx_buf.at[j]],
                             ssem[kk], add=True)
            pltpu.make_async_copy(
                e_buf.at[kp], acc.at[idx_buf.at[0]], ssem[kp]).wait()
        pltpu.make_async_copy(
            e_buf.at[1], acc.at[idx_buf.at[0]], ssem[1]).wait()

        plsc.subcore_barrier()
        @pl.when(s < IO_TILES)
        def _():
            pltpu.sync_copy(
                acc.at[pl.ds(s * IO_ROWS, IO_ROWS)],
                out_hbm.at[c, pl.ds(s * IO_ROWS, IO_ROWS)])

    return k(e, idxA)


def _dot_nt(a, w):
    return lax.dot_general(a, w, (((1,), (1,)), ((), ())),
                           preferred_element_type=jnp.float32)


def _mlp_body(p_ref, v_ref, w1_ref, b1_ref, w2_ref, b2_ref, wg_ref,
              bg_ref, o_ref):
    shift = jnp.float32(0.6931471805599453)
    su = p_ref[0] + p_ref[1]
    z1 = _dot_nt(su, w1_ref[...]) + b1_ref[...]
    h1 = jnp.maximum(z1, 0.0) + jnp.log1p(jnp.exp(-jnp.abs(z1))) - shift
    h2 = _dot_nt(h1, w2_ref[...]) + b2_ref[...]
    vv = v_ref[...]
    zg = (_dot_nt(vv, wg_ref[:, :H]) + _dot_nt(h2, wg_ref[:, H:])
          + bg_ref[...])
    g = 1.0 / (1.0 + jnp.exp(-zg))
    o_ref[...] = g * vv + (1.0 - g) * h2


def _mlp_update_tc(partials, v, W1, b1, W2, b2, Wg, bg):
    blk = 5000
    grid = N // blk
    full = lambda shape: pl.BlockSpec(shape, lambda i: (0,) * len(shape))
    return pl.pallas_call(
        _mlp_body,
        grid=(grid,),
        in_specs=[
            pl.BlockSpec((2, blk, H), lambda i: (0, i, 0)),
            pl.BlockSpec((blk, H), lambda i: (i, 0)),
            full((H, H)), full((1, H)),
            full((H, H)), full((1, H)),
            full((H, 2 * H)), full((1, H)),
        ],
        out_specs=pl.BlockSpec((blk, H), lambda i: (i, 0)),
        out_shape=jax.ShapeDtypeStruct((N, H), jnp.float32),
    )(partials, v, W1, b1, W2, b2, Wg, bg)


def kernel(v, e, edge_index, W1, b1, W2, b2, Wg, bg):
    ei = edge_index
    if ei.dtype != jnp.int32:
        ei = lax.bitcast_convert_type(ei, jnp.int32)[..., 0]
    idxA = ei.reshape(2, NW, NCHM, CHM)
    partials = _segment_sum_sc(e, idxA)
    return _mlp_update_tc(
        partials, v,
        W1, b1.reshape(1, H),
        W2, b2.reshape(1, H),
        Wg, bg.reshape(1, H),
    )


# speedup vs baseline: 1.0172x; 1.0017x over previous
import functools

import jax
import jax.numpy as jnp
from jax import lax
from jax.experimental import pallas as pl
from jax.experimental.pallas import tpu as pltpu
from jax.experimental.pallas import tpu_sc as plsc

N = 10000
E = 320000
H = 128

NW = 32
PERW = E // NW
CHM = 80
NCHM = PERW // CHM
NBUF = 3
IO_ROWS = 624
IO_LAST = N - 15 * IO_ROWS
ZROWS = 48


def _segment_sum_sc(e, idxA):
    mesh = plsc.VectorSubcoreMesh(core_axis_name="c", subcore_axis_name="s")

    @functools.partial(
        pl.kernel,
        out_type=jax.ShapeDtypeStruct((2, N, H), jnp.float32),
        mesh=mesh,
        scratch_types=[
            pltpu.VMEM((NBUF, CHM, H), jnp.float32),
            pltpu.VMEM((NCHM, CHM), jnp.int32),
            pltpu.VMEM_SHARED((N, H), jnp.float32),
        ] + [pltpu.SemaphoreType.DMA] * (2 * NBUF),
    )
    def k(e_hbm, idxA_hbm, out_hbm, e_buf, idx_buf, acc, *sems):
        lsem = sems[:NBUF]
        ssem = sems[NBUF:]
        c = lax.axis_index("c")
        s = lax.axis_index("s")
        wid = s * 2 + c
        base = wid * PERW

        for j in range(NBUF - 1):
            pltpu.async_copy(e_hbm.at[pl.ds(base + j * CHM, CHM)],
                             e_buf.at[j], lsem[j])
        pltpu.sync_copy(idxA_hbm.at[1, wid], idx_buf)

        z16 = jnp.zeros((16,), jnp.float32)

        def zrow(r, carry):
            for q in range(H // 16):
                e_buf[2, r, pl.ds(q * 16, 16)] = z16
            return carry

        lax.fori_loop(0, ZROWS, zrow, 0)
        for i in range(IO_ROWS // ZROWS):
            pltpu.sync_copy(
                e_buf.at[2, pl.ds(0, ZROWS)],
                acc.at[pl.ds(s * IO_ROWS + i * ZROWS, ZROWS)])

        @pl.when(s == 15)
        def _():
            pltpu.sync_copy(
                e_buf.at[2, pl.ds(0, IO_LAST - IO_ROWS)],
                acc.at[pl.ds(16 * IO_ROWS, IO_LAST - IO_ROWS)])
        plsc.subcore_barrier()

        def body(g, carry):
            for kk in range(NBUF):
                j = NBUF * g + kk
                kp = (kk + 2) % NBUF
                pltpu.make_async_copy(
                    e_hbm.at[pl.ds(base, CHM)], e_buf.at[kk], lsem[kk]).wait()
                pltpu.async_copy(e_buf.at[kk], acc.at[idx_buf.at[j]],
                                 ssem[kk], add=True)

                @pl.when(j >= 1)
                def _():
                    pltpu.make_async_copy(
                        e_buf.at[kp], acc.at[idx_buf.at[0]], ssem[kp]).wait()

                pltpu.async_copy(
                    e_hbm.at[pl.ds(base + (j + 2) * CHM, CHM)],
                    e_buf.at[kp], lsem[kp])
            return carry

        G = (NCHM - 2) // NBUF
        lax.fori_loop(0, G, body, 0)
        for kk in range(NCHM - NBUF * G):
            j = NBUF * G + kk
            kp = (kk + 2) % NBUF
            pltpu.make_async_copy(
                e_hbm.at[pl.ds(base, CHM)], e_buf.at[kk], lsem[kk]).wait()
            pltpu.async_copy(e_buf.at[kk], acc.at[idx_buf.at[j]],
                             ssem[kk], add=True)
            pltpu.make_async_copy(
                e_buf.at[kp], acc.at[idx_buf.at[0]], ssem[kp]).wait()
        pltpu.make_async_copy(
            e_buf.at[1], acc.at[idx_buf.at[0]], ssem[1]).wait()

        plsc.subcore_barrier()
        pltpu.sync_copy(
            acc.at[pl.ds(s * IO_ROWS, IO_ROWS)],
            out_hbm.at[c, pl.ds(s * IO_ROWS, IO_ROWS)])

        @pl.when(s == 15)
        def _():
            pltpu.sync_copy(
                acc.at[pl.ds(16 * IO_ROWS, IO_LAST - IO_ROWS)],
                out_hbm.at[c, pl.ds(16 * IO_ROWS, IO_LAST - IO_ROWS)])

    return k(e, idxA)


def _dot_nt(a, w):
    return lax.dot_general(a, w, (((1,), (1,)), ((), ())),
                           preferred_element_type=jnp.float32)


def _mlp_body(p_ref, v_ref, w1_ref, b1_ref, w2_ref, b2_ref, wg_ref,
              bg_ref, o_ref):
    shift = jnp.float32(0.6931471805599453)
    su = p_ref[0] + p_ref[1]
    z1 = _dot_nt(su, w1_ref[...]) + b1_ref[...]
    h1 = jnp.maximum(z1, 0.0) + jnp.log1p(jnp.exp(-jnp.abs(z1))) - shift
    h2 = _dot_nt(h1, w2_ref[...]) + b2_ref[...]
    vv = v_ref[...]
    zg = (_dot_nt(vv, wg_ref[:, :H]) + _dot_nt(h2, wg_ref[:, H:])
          + bg_ref[...])
    g = 1.0 / (1.0 + jnp.exp(-zg))
    o_ref[...] = g * vv + (1.0 - g) * h2


def _mlp_update_tc(partials, v, W1, b1, W2, b2, Wg, bg):
    blk = 5000
    grid = N // blk
    full = lambda shape: pl.BlockSpec(shape, lambda i: (0,) * len(shape))
    return pl.pallas_call(
        _mlp_body,
        grid=(grid,),
        in_specs=[
            pl.BlockSpec((2, blk, H), lambda i: (0, i, 0)),
            pl.BlockSpec((blk, H), lambda i: (i, 0)),
            full((H, H)), full((1, H)),
            full((H, H)), full((1, H)),
            full((H, 2 * H)), full((1, H)),
        ],
        out_specs=pl.BlockSpec((blk, H), lambda i: (i, 0)),
        out_shape=jax.ShapeDtypeStruct((N, H), jnp.float32),
    )(partials, v, W1, b1, W2, b2, Wg, bg)


def kernel(v, e, edge_index, W1, b1, W2, b2, Wg, bg):
    ei = edge_index
    if ei.dtype != jnp.int32:
        ei = lax.bitcast_convert_type(ei, jnp.int32)[..., 0]
    idxA = ei.reshape(2, NW, NCHM, CHM)
    partials = _segment_sum_sc(e, idxA)
    return _mlp_update_tc(
        partials, v,
        W1, b1.reshape(1, H),
        W2, b2.reshape(1, H),
        Wg, bg.reshape(1, H),
    )
